# baseline (speedup 1.0000x reference)
import functools

import jax
import jax.numpy as jnp
from jax import lax
from jax.experimental import pallas as pl
from jax.experimental.pallas import tpu as pltpu
from jax.experimental.pallas import tpu_sc as plsc

N = 10000
E = 5000
NNZ = 320000
D = 128

NC = 2
NS = 16
L = 16
NW = NC * NS
DH = D // 2
W = DH + L
NV = W // L

N_PAD = 10240
E_PAD = 5120
BATCH = 128
NB = 160
NNZ_PAD = NS * NB * BATCH

_mesh = plsc.VectorSubcoreMesh(
    core_axis_name="c", subcore_axis_name="s", num_cores=NC, num_subcores=NS
)
_sc_params = pltpu.CompilerParams(use_tc_tiling_on_sc=False)


def _worker_id():
    return lax.axis_index("s") * NC + lax.axis_index("c")


def _mm_body(x_ref, t_ref, o_ref):
    o_ref[...] = jnp.dot(x_ref[...], t_ref[...],
                         preferred_element_type=jnp.float32)


def _matmul(x_pad, theta):
    bm = 256
    return pl.pallas_call(
        _mm_body,
        grid=(N_PAD // bm,),
        in_specs=[
            pl.BlockSpec((bm, D), lambda i: (i, 0)),
            pl.BlockSpec((D, D), lambda i: (0, 0)),
        ],
        out_specs=pl.BlockSpec((bm, D), lambda i: (i, 0)),
        out_shape=jax.ShapeDtypeStruct((N_PAD, D), jnp.float32),
    )(x_pad, theta)


NBUF = 2


def _make_pass(acc_rows):
    rows_per_sub = acc_rows // NS
    zr = 160

    @functools.partial(
        pl.kernel,
        out_type=jax.ShapeDtypeStruct((NC, acc_rows, W), jnp.float32),
        mesh=_mesh,
        scratch_types=[
            pltpu.VMEM((NB, BATCH), jnp.int32),
            pltpu.VMEM((NB, BATCH), jnp.int32),
            pltpu.VMEM((NBUF, BATCH, W), jnp.float32),
            pltpu.VMEM((zr, W), jnp.float32),
            pltpu.VMEM_SHARED((acc_rows, W), jnp.float32),
        ] + [pltpu.SemaphoreType.DMA] * (2 * NBUF),
        compiler_params=_sc_params,
    )
    def k(tab_hbm, gidx_hbm, sidx_hbm, out_hbm,
          gidx_v, sidx_v, buf, zbuf, acc, *sems):
        cid = lax.axis_index("c")
        sid = lax.axis_index("s")
        gsem, ssem = sems[:NBUF], sems[NBUF:]

        def zrow(r, carry):
            for j in range(NV):
                zbuf[r, pl.ds(j * L, L)] = jnp.zeros((L,), jnp.float32)
            return carry
        lax.fori_loop(0, zr, zrow, 0)
        for cpy in range(rows_per_sub // zr):
            pltpu.sync_copy(
                zbuf, acc.at[pl.ds(sid * rows_per_sub + cpy * zr, zr)])

        pltpu.sync_copy(gidx_hbm.at[cid, sid], gidx_v)
        pltpu.sync_copy(sidx_hbm.at[sid], sidx_v)
        plsc.subcore_barrier()

        def start_g(j, b):
            pltpu.async_copy(tab_hbm.at[gidx_v.at[j]], buf.at[b], gsem[b])

        def wait_g(b):
            pltpu.make_async_copy(tab_hbm.at[gidx_v.at[0]], buf.at[b],
                                  gsem[b]).wait()

        def start_s(j, b):
            pltpu.async_copy(buf.at[b], acc.at[sidx_v.at[j]], ssem[b],
                             add=True)

        def wait_s(b):
            pltpu.make_async_copy(buf.at[b], acc.at[sidx_v.at[0]],
                                  ssem[b]).wait()

        for b in range(NBUF):
            start_g(b, b)

        def body(i, carry):
            for b in range(NBUF):
                wait_g(b)
                start_s(NBUF * i + b, b)

            @pl.when(i < NB // NBUF - 1)
            def _():
                for b in range(NBUF):
                    wait_s(b)
                    start_g(NBUF * (i + 1) + b, b)
            return carry
        lax.fori_loop(0, NB // NBUF, body, 0)
        for b in range(NBUF):
            wait_s(b)

        plsc.subcore_barrier()
        pltpu.sync_copy(
            acc.at[pl.ds(sid * rows_per_sub, rows_per_sub)],
            out_hbm.at[cid, pl.ds(sid * rows_per_sub, rows_per_sub)])

    return k


_pass_a = _make_pass(E_PAD)
_pass_b = _make_pass(N_PAD)


@functools.partial(
    pl.kernel,
    out_type=jax.ShapeDtypeStruct((NC * E_PAD, W), jnp.float32),
    mesh=_mesh,
    scratch_types=[pltpu.VMEM((160, W), jnp.float32)],
    compiler_params=_sc_params,
)
def _scale(part_hbm, out_hbm, buf):
    wid = _worker_id()
    for half in range(2):
        base = wid * 320 + half * 160
        pltpu.sync_copy(part_hbm.at[pl.ds(base, 160)], buf)

        def row(r, carry):
            s = [buf[r, pl.ds(j * L, L)] for j in range(NV)]
            cnt = s[NV - 1]
            pos = cnt > 0.0
            norm = 1.0 / jnp.where(pos, cnt, 1.0)
            for j in range(NV - 1):
                buf[r, pl.ds(j * L, L)] = s[j] * norm
            buf[r, pl.ds(DH, L)] = jnp.where(pos, 1.0, 0.0)
            return carry
        lax.fori_loop(0, 160, row, 0)
        pltpu.sync_copy(buf, out_hbm.at[pl.ds(base, 160)])


@functools.partial(
    pl.kernel,
    out_type=jax.ShapeDtypeStruct((N_PAD, D), jnp.float32),
    mesh=_mesh,
    scratch_types=[
        pltpu.VMEM((160, W), jnp.float32),
        pltpu.VMEM((160, W), jnp.float32),
        pltpu.VMEM((160, D), jnp.float32),
        pltpu.VMEM((D,), jnp.float32),
    ],
    compiler_params=_sc_params,
)
def _final(part_hbm, bias_hbm, out_hbm, buf_l, buf_h, buf_o, bias_v):
    wid = _worker_id()
    pltpu.sync_copy(bias_hbm, bias_v)
    for half in range(2):
        base = wid * 320 + half * 160
        pltpu.sync_copy(part_hbm.at[0, pl.ds(base, 160)], buf_l)
        pltpu.sync_copy(part_hbm.at[1, pl.ds(base, 160)], buf_h)

        def row(r, carry):
            cnt = buf_l[r, pl.ds(DH, L)]
            pos = cnt > 0.0
            norm = 1.0 / jnp.where(pos, cnt, 1.0)
            norm = jnp.where(pos, norm, 0.0)
            for j in range(NV - 1):
                buf_o[r, pl.ds(j * L, L)] = (
                    buf_l[r, pl.ds(j * L, L)] * norm
                    + bias_v[pl.ds(j * L, L)])
                buf_o[r, pl.ds(DH + j * L, L)] = (
                    buf_h[r, pl.ds(j * L, L)] * norm
                    + bias_v[pl.ds(DH + j * L, L)])
            return carry
        lax.fori_loop(0, 160, row, 0)
        pltpu.sync_copy(buf_o, out_hbm.at[pl.ds(base, 160)])


def kernel(X, X_target, theta, bias, H_row, H_col):
    del X_target
    x_pad = jnp.zeros((N_PAD, D), jnp.float32).at[:N].set(X)
    xp = _matmul(x_pad, theta)
    ones = jnp.ones((N_PAD, L), jnp.float32)
    xp2 = jnp.concatenate([
        jnp.concatenate([xp[:, :DH], ones], axis=1),
        jnp.concatenate([xp[:, DH:], ones], axis=1),
    ], axis=0)

    pad_n = NNZ_PAD - NNZ
    hr = jnp.concatenate(
        [H_row, jnp.full((pad_n,), N_PAD - 1, jnp.int32)]
    ).reshape(NS, NB, BATCH)
    hc = jnp.concatenate(
        [H_col, jnp.full((pad_n,), E_PAD - 1, jnp.int32)]
    ).reshape(NS, NB, BATCH)
    hr_g = jnp.stack([hr, hr + N_PAD])
    hc_g = jnp.stack([hc, hc + E_PAD])

    y_part = _pass_a(xp2, hr_g, hc)
    y_aug = _scale(y_part.reshape(NC * E_PAD, W))
    x_part = _pass_b(y_aug, hc_g, hr)
    out = _final(x_part, bias)
    return out[:N]


# speedup vs baseline: 1.2178x; 1.2178x over previous
import functools

import jax
import jax.numpy as jnp
from jax import lax
from jax.experimental import pallas as pl
from jax.experimental.pallas import tpu as pltpu
from jax.experimental.pallas import tpu_sc as plsc

N = 10000
E = 5000
NNZ = 320000
D = 128

NC = 2
NS = 16
L = 16
DH = D // 2
W = DH + L
NV = W // L

N_PAD = 10240
E_PAD = 5120
BATCH = 128
NB = 160
NBS = 40
NSTG = NB // NBS
NNZ_PAD = NS * NB * BATCH

_mesh = plsc.VectorSubcoreMesh(
    core_axis_name="c", subcore_axis_name="s", num_cores=NC, num_subcores=NS
)
_sc_params = pltpu.CompilerParams(use_tc_tiling_on_sc=False)


def _mm_body(x_ref, t_ref, o_ref):
    o_ref[...] = jnp.dot(x_ref[...], t_ref[...],
                         preferred_element_type=jnp.float32)


def _matmul(x_pad, theta):
    bm = 256
    return pl.pallas_call(
        _mm_body,
        grid=(N_PAD // bm,),
        in_specs=[
            pl.BlockSpec((bm, D), lambda i: (i, 0)),
            pl.BlockSpec((D, D), lambda i: (0, 0)),
        ],
        out_specs=pl.BlockSpec((bm, D), lambda i: (i, 0)),
        out_shape=jax.ShapeDtypeStruct((N_PAD, D), jnp.float32),
    )(x_pad, theta)


@functools.partial(
    pl.kernel,
    out_type=(jax.ShapeDtypeStruct((N_PAD, D), jnp.float32),
              jax.ShapeDtypeStruct((NC * E_PAD, W), jnp.float32)),
    mesh=_mesh,
    scratch_types=[
        pltpu.VMEM((NBS, BATCH), jnp.int32),
        pltpu.VMEM((NBS, BATCH), jnp.int32),
        pltpu.VMEM((2, BATCH, W), jnp.float32),
        pltpu.VMEM((80, W), jnp.float32),
        pltpu.VMEM((80, DH), jnp.float32),
        pltpu.VMEM((DH,), jnp.float32),
        pltpu.VMEM_SHARED((E_PAD, W), jnp.float32),
        pltpu.VMEM_SHARED((N_PAD, W), jnp.float32),
        pltpu.SemaphoreType.DMA,
        pltpu.SemaphoreType.DMA,
    ],
    compiler_params=_sc_params,
)
def _sc_fused(xp2_hbm, hrg_hbm, hcg_hbm, hr_hbm, hc_hbm, bias_hbm,
              out_hbm, yaug_hbm,
              gidx_v, sidx_v, buf, sbuf, pbuf, bias_v, yacc, xacc,
              sem0, sem1):
    cid = lax.axis_index("c")
    sid = lax.axis_index("s")
    sems = (sem0, sem1)

    def zrow(r, carry):
        for j in range(NV):
            sbuf[r, pl.ds(j * L, L)] = jnp.zeros((L,), jnp.float32)
        return carry
    lax.fori_loop(0, 80, zrow, 0)
    for cpy in range(4):
        pltpu.sync_copy(sbuf, yacc.at[pl.ds(sid * 320 + cpy * 80, 80)])
    for cpy in range(8):
        pltpu.sync_copy(sbuf, xacc.at[pl.ds(sid * 640 + cpy * 80, 80)])
    pltpu.sync_copy(bias_hbm.at[pl.ds(cid * DH, DH)], bias_v)
    plsc.subcore_barrier()

    def run_pass(tab, g_hbm, s_hbm, acc):
        def start(j, slot):
            pltpu.async_copy(tab.at[gidx_v.at[j]], buf.at[slot], sems[slot])

        def wait(slot):
            pltpu.make_async_copy(tab.at[gidx_v.at[0]], buf.at[slot],
                                  sems[slot]).wait()

        for st in range(NSTG):
            pltpu.sync_copy(g_hbm.at[pl.ds(st * NBS, NBS)], gidx_v)
            pltpu.sync_copy(s_hbm.at[pl.ds(st * NBS, NBS)], sidx_v)
            start(0, 0)

            def body(i, carry):
                j0 = 2 * i
                start(j0 + 1, 1)
                wait(0)
                pltpu.sync_copy(buf.at[0], acc.at[sidx_v.at[j0]], add=True)

                @pl.when(i < NBS // 2 - 1)
                def _():
                    start(j0 + 2, 0)

                wait(1)
                pltpu.sync_copy(buf.at[1], acc.at[sidx_v.at[j0 + 1]],
                                add=True)
                return carry
            lax.fori_loop(0, NBS // 2, body, 0)

    run_pass(xp2_hbm, hrg_hbm.at[cid, sid], hc_hbm.at[sid], yacc)
    plsc.subcore_barrier()

    for chunk in range(4):
        rows = sid * 320 + chunk * 80
        pltpu.sync_copy(yacc.at[pl.ds(rows, 80)], sbuf)

        def yrow(r, carry):
            s = [sbuf[r, pl.ds(j * L, L)] for j in range(NV)]
            cnt = s[NV - 1]
            pos = cnt > 0.0
            norm = 1.0 / jnp.where(pos, cnt, 1.0)
            for j in range(NV - 1):
                sbuf[r, pl.ds(j * L, L)] = s[j] * norm
            sbuf[r, pl.ds(DH, L)] = jnp.where(pos, 1.0, 0.0)
            return carry
        lax.fori_loop(0, 80, yrow, 0)
        pltpu.sync_copy(sbuf, yaug_hbm.at[pl.ds(cid * E_PAD + rows, 80)])
    plsc.subcore_barrier()

    run_pass(yaug_hbm, hcg_hbm.at[cid, sid], hr_hbm.at[sid], xacc)
    plsc.subcore_barrier()

    for chunk in range(8):
        rows = sid * 640 + chunk * 80
        pltpu.sync_copy(xacc.at[pl.ds(rows, 80)], sbuf)

        def xrow(r, carry):
            cnt = sbuf[r, pl.ds(DH, L)]
            pos = cnt > 0.0
            norm = 1.0 / jnp.where(pos, cnt, 1.0)
            norm = jnp.where(pos, norm, 0.0)
            for j in range(NV - 1):
                pbuf[r, pl.ds(j * L, L)] = (
                    sbuf[r, pl.ds(j * L, L)] * norm + bias_v[pl.ds(j * L, L)])
            return carry
        lax.fori_loop(0, 80, xrow, 0)
        pltpu.sync_copy(
            pbuf, out_hbm.at[pl.ds(rows, 80), pl.ds(cid * DH, DH)])


def kernel(X, X_target, theta, bias, H_row, H_col):
    del X_target
    x_pad = jnp.zeros((N_PAD, D), jnp.float32).at[:N].set(X)
    xp = _matmul(x_pad, theta)
    ones = jnp.ones((N_PAD, L), jnp.float32)
    xp2 = jnp.concatenate([
        jnp.concatenate([xp[:, :DH], ones], axis=1),
        jnp.concatenate([xp[:, DH:], ones], axis=1),
    ], axis=0)

    pad_n = NNZ_PAD - NNZ
    hr = jnp.concatenate(
        [H_row, jnp.full((pad_n,), N_PAD - 1, jnp.int32)]
    ).reshape(NS, NB, BATCH)
    hc = jnp.concatenate(
        [H_col, jnp.full((pad_n,), E_PAD - 1, jnp.int32)]
    ).reshape(NS, NB, BATCH)
    hr_g = jnp.stack([hr, hr + N_PAD])
    hc_g = jnp.stack([hc, hc + E_PAD])

    out, _ = _sc_fused(xp2, hr_g, hc_g, hr, hc, bias)
    return out[:N]


# speedup vs baseline: 1.2401x; 1.0183x over previous
import functools

import jax
import jax.numpy as jnp
from jax import lax
from jax.experimental import pallas as pl
from jax.experimental.pallas import tpu as pltpu
from jax.experimental.pallas import tpu_sc as plsc

N = 10000
E = 5000
NNZ = 320000
D = 128

NC = 2
NS = 16
L = 16
DH = D // 2
W = DH + L
NV = W // L

N_PAD = 10240
E_PAD = 5120
BATCH = 128
NB = 160
NBS = 20
NSTG = NB // NBS
NBUF = 4
NNZ_PAD = NS * NB * BATCH

_mesh = plsc.VectorSubcoreMesh(
    core_axis_name="c", subcore_axis_name="s", num_cores=NC, num_subcores=NS
)
_sc_params = pltpu.CompilerParams(use_tc_tiling_on_sc=False)


def _mm_body(x_ref, t_ref, o_ref):
    o_ref[...] = jnp.dot(x_ref[...], t_ref[...],
                         preferred_element_type=jnp.float32)


def _matmul(x_pad, theta):
    bm = 256
    return pl.pallas_call(
        _mm_body,
        grid=(N_PAD // bm,),
        in_specs=[
            pl.BlockSpec((bm, D), lambda i: (i, 0)),
            pl.BlockSpec((D, D), lambda i: (0, 0)),
        ],
        out_specs=pl.BlockSpec((bm, D), lambda i: (i, 0)),
        out_shape=jax.ShapeDtypeStruct((N_PAD, D), jnp.float32),
    )(x_pad, theta)


@functools.partial(
    pl.kernel,
    out_type=(jax.ShapeDtypeStruct((N_PAD, D), jnp.float32),
              jax.ShapeDtypeStruct((NC * E_PAD, W), jnp.float32)),
    mesh=_mesh,
    scratch_types=[
        pltpu.VMEM((NBS, BATCH), jnp.int32),
        pltpu.VMEM((NBS, BATCH), jnp.int32),
        pltpu.VMEM((NBUF, BATCH, W), jnp.float32),
        pltpu.VMEM((40, W), jnp.float32),
        pltpu.VMEM((40, DH), jnp.float32),
        pltpu.VMEM((DH,), jnp.float32),
        pltpu.VMEM_SHARED((E_PAD, W), jnp.float32),
        pltpu.VMEM_SHARED((N_PAD, W), jnp.float32),
    ] + [pltpu.SemaphoreType.DMA] * (2 * NBUF),
    compiler_params=_sc_params,
)
def _sc_fused(xp2_hbm, hrg_hbm, hcg_hbm, hr_hbm, hc_hbm, bias_hbm,
              out_hbm, yaug_hbm,
              gidx_v, sidx_v, buf, sbuf, pbuf, bias_v, yacc, xacc,
              *sems):
    cid = lax.axis_index("c")
    sid = lax.axis_index("s")
    gsem, ssem = sems[:NBUF], sems[NBUF:]

    def zrow(r, carry):
        for j in range(NV):
            sbuf[r, pl.ds(j * L, L)] = jnp.zeros((L,), jnp.float32)
        return carry
    lax.fori_loop(0, 40, zrow, 0)
    for cpy in range(8):
        pltpu.sync_copy(sbuf, yacc.at[pl.ds(sid * 320 + cpy * 40, 40)])
    for cpy in range(16):
        pltpu.sync_copy(sbuf, xacc.at[pl.ds(sid * 640 + cpy * 40, 40)])
    pltpu.sync_copy(bias_hbm.at[pl.ds(cid * DH, DH)], bias_v)
    plsc.subcore_barrier()

    def run_pass(tab, g_hbm, s_hbm, acc):
        def start_g(j, b):
            pltpu.async_copy(tab.at[gidx_v.at[j]], buf.at[b], gsem[b])

        def wait_g(b):
            pltpu.make_async_copy(tab.at[gidx_v.at[0]], buf.at[b],
                                  gsem[b]).wait()

        def start_s(j, b):
            pltpu.async_copy(buf.at[b], acc.at[sidx_v.at[j]], ssem[b],
                             add=True)

        def wait_s(b):
            pltpu.make_async_copy(buf.at[b], acc.at[sidx_v.at[0]],
                                  ssem[b]).wait()

        for st in range(NSTG):
            pltpu.sync_copy(g_hbm.at[pl.ds(st * NBS, NBS)], gidx_v)
            pltpu.sync_copy(s_hbm.at[pl.ds(st * NBS, NBS)], sidx_v)
            for b in range(NBUF):
                start_g(b, b)

            def body(i, carry):
                for b in range(NBUF):
                    wait_g(b)
                    start_s(NBUF * i + b, b)

                @pl.when(i < NBS // NBUF - 1)
                def _():
                    for b in range(NBUF):
                        wait_s(b)
                        start_g(NBUF * (i + 1) + b, b)
                return carry
            lax.fori_loop(0, NBS // NBUF, body, 0)
            for b in range(NBUF):
                wait_s(b)

    run_pass(xp2_hbm, hrg_hbm.at[cid, sid], hc_hbm.at[sid], yacc)
    plsc.subcore_barrier()

    for chunk in range(8):
        rows = sid * 320 + chunk * 40
        pltpu.sync_copy(yacc.at[pl.ds(rows, 40)], sbuf)

        def yrow(r, carry):
            s = [sbuf[r, pl.ds(j * L, L)] for j in range(NV)]
            cnt = s[NV - 1]
            pos = cnt > 0.0
            norm = 1.0 / jnp.where(pos, cnt, 1.0)
            for j in range(NV - 1):
                sbuf[r, pl.ds(j * L, L)] = s[j] * norm
            sbuf[r, pl.ds(DH, L)] = jnp.where(pos, 1.0, 0.0)
            return carry
        lax.fori_loop(0, 40, yrow, 0)
        pltpu.sync_copy(sbuf, yaug_hbm.at[pl.ds(cid * E_PAD + rows, 40)])
    plsc.subcore_barrier()

    run_pass(yaug_hbm, hcg_hbm.at[cid, sid], hr_hbm.at[sid], xacc)
    plsc.subcore_barrier()

    for chunk in range(16):
        rows = sid * 640 + chunk * 40
        pltpu.sync_copy(xacc.at[pl.ds(rows, 40)], sbuf)

        def xrow(r, carry):
            cnt = sbuf[r, pl.ds(DH, L)]
            pos = cnt > 0.0
            norm = 1.0 / jnp.where(pos, cnt, 1.0)
            norm = jnp.where(pos, norm, 0.0)
            for j in range(NV - 1):
                pbuf[r, pl.ds(j * L, L)] = (
                    sbuf[r, pl.ds(j * L, L)] * norm + bias_v[pl.ds(j * L, L)])
            return carry
        lax.fori_loop(0, 40, xrow, 0)
        pltpu.sync_copy(
            pbuf, out_hbm.at[pl.ds(rows, 40), pl.ds(cid * DH, DH)])


def kernel(X, X_target, theta, bias, H_row, H_col):
    del X_target
    x_pad = jnp.zeros((N_PAD, D), jnp.float32).at[:N].set(X)
    xp = _matmul(x_pad, theta)
    ones = jnp.ones((N_PAD, L), jnp.float32)
    xp2 = jnp.concatenate([
        jnp.concatenate([xp[:, :DH], ones], axis=1),
        jnp.concatenate([xp[:, DH:], ones], axis=1),
    ], axis=0)

    pad_n = NNZ_PAD - NNZ
    hr = jnp.concatenate(
        [H_row, jnp.full((pad_n,), N_PAD - 1, jnp.int32)]
    ).reshape(NS, NB, BATCH)
    hc = jnp.concatenate(
        [H_col, jnp.full((pad_n,), E_PAD - 1, jnp.int32)]
    ).reshape(NS, NB, BATCH)
    hr_g = jnp.stack([hr, hr + N_PAD])
    hc_g = jnp.stack([hc, hc + E_PAD])

    out, _ = _sc_fused(xp2, hr_g, hc_g, hr, hc, bias)
    return out[:N]


# speedup vs baseline: 1.3649x; 1.1006x over previous
import functools

import jax
import jax.numpy as jnp
from jax import lax
from jax.experimental import pallas as pl
from jax.experimental.pallas import tpu as pltpu
from jax.experimental.pallas import tpu_sc as plsc

N = 10000
E = 5000
NNZ = 320000
D = 128

NC = 2
NS = 16
L = 16
DH = D // 2
W = DH
NV = W // L

N_PAD = 10240
E_PAD = 5120
BATCH = 128
NB = 160
NBS = 20
NSTG = NB // NBS
NBUF = 2
NNZ_PAD = NS * NB * BATCH
ER = E_PAD // NS
NR = N_PAD // NS

_mesh = plsc.VectorSubcoreMesh(
    core_axis_name="c", subcore_axis_name="s", num_cores=NC, num_subcores=NS
)
_sc_params = pltpu.CompilerParams(use_tc_tiling_on_sc=False,
                                  needs_layout_passes=False)


def _mm_body(x_ref, t_ref, o_ref):
    o_ref[...] = jnp.dot(x_ref[...], t_ref[...],
                         preferred_element_type=jnp.float32)


def _matmul(x_pad, theta):
    bm = 256
    return pl.pallas_call(
        _mm_body,
        grid=(N_PAD // bm,),
        in_specs=[
            pl.BlockSpec((bm, D), lambda i: (i, 0)),
            pl.BlockSpec((D, D), lambda i: (0, 0)),
        ],
        out_specs=pl.BlockSpec((bm, D), lambda i: (i, 0)),
        out_shape=jax.ShapeDtypeStruct((N_PAD, D), jnp.float32),
    )(x_pad, theta)


@functools.partial(
    pl.kernel,
    out_type=(jax.ShapeDtypeStruct((N_PAD, D), jnp.float32),
              jax.ShapeDtypeStruct((NC * E_PAD, W), jnp.float32)),
    mesh=_mesh,
    scratch_types=[
        pltpu.VMEM((NBS, BATCH), jnp.int32),
        pltpu.VMEM((NBS, BATCH), jnp.int32),
        pltpu.VMEM((NBUF, BATCH, W), jnp.float32),
        pltpu.VMEM((40, W), jnp.float32),
        pltpu.VMEM((40, DH), jnp.float32),
        pltpu.VMEM((DH,), jnp.float32),
        pltpu.VMEM((E_PAD,), jnp.float32),
        pltpu.VMEM((N_PAD,), jnp.float32),
        pltpu.VMEM((NS, NR), jnp.float32),
        pltpu.VMEM((NR,), jnp.float32),
        pltpu.VMEM_SHARED((E_PAD, W), jnp.float32),
        pltpu.VMEM_SHARED((N_PAD, W), jnp.float32),
        pltpu.VMEM_SHARED((NS, N_PAD), jnp.float32),
        pltpu.SemaphoreType.DMA,
        pltpu.SemaphoreType.DMA,
        pltpu.SemaphoreType.DMA,
        pltpu.SemaphoreType.DMA,
    ],
    compiler_params=_sc_params,
)
def _sc_fused(xp2_hbm, hrg_hbm, hcg_hbm, hr_hbm, hc_hbm, bias_hbm,
              out_hbm, yaug_hbm,
              gidx_v, sidx_v, buf, sbuf, pbuf, bias_v, cnt_e, cnt_n,
              redbuf, cred, yacc, xacc, cstage,
              sem0, sem1, sem2, sem3):
    cid = lax.axis_index("c")
    sid = lax.axis_index("s")
    gsem = (sem0, sem1)
    ssem = (sem2, sem3)
    ones_v = jnp.full((L,), 1.0, jnp.float32)
    zeros_v = jnp.zeros((L,), jnp.float32)

    def zrow(r, carry):
        for j in range(NV):
            sbuf[r, pl.ds(j * L, L)] = zeros_v
        return carry
    lax.fori_loop(0, 40, zrow, 0)
    for cpy in range(ER // 40):
        pltpu.sync_copy(sbuf, yacc.at[pl.ds(sid * ER + cpy * 40, 40)])
    for cpy in range(NR // 40):
        pltpu.sync_copy(sbuf, xacc.at[pl.ds(sid * NR + cpy * 40, 40)])

    def zv_e(i, carry):
        cnt_e[pl.ds(i * L, L)] = zeros_v
        return carry
    lax.fori_loop(0, E_PAD // L, zv_e, 0)

    def zv_n(i, carry):
        cnt_n[pl.ds(i * L, L)] = zeros_v
        return carry
    lax.fori_loop(0, N_PAD // L, zv_n, 0)

    pltpu.sync_copy(bias_hbm.at[pl.ds(cid * DH, DH)], bias_v)
    plsc.subcore_barrier()

    def run_pass(tab, g_hbm, s_hbm, acc, cnt):
        def start_g(j, b):
            pltpu.async_copy(tab.at[gidx_v.at[j]], buf.at[b], gsem[b])

        def wait_g(b):
            pltpu.make_async_copy(tab.at[gidx_v.at[0]], buf.at[b],
                                  gsem[b]).wait()

        def start_s(j, b):
            pltpu.async_copy(buf.at[b], acc.at[sidx_v.at[j]], ssem[b],
                             add=True)

        def wait_s(b):
            pltpu.make_async_copy(buf.at[b], acc.at[sidx_v.at[0]],
                                  ssem[b]).wait()

        for st in range(NSTG):
            pltpu.sync_copy(g_hbm.at[pl.ds(st * NBS, NBS)], gidx_v)
            pltpu.sync_copy(s_hbm.at[pl.ds(st * NBS, NBS)], sidx_v)
            for b in range(NBUF):
                start_g(b, b)

            def body(i, carry):
                for b in range(NBUF):
                    j = NBUF * i + b
                    wait_g(b)
                    start_s(j, b)
                    for v in range(BATCH // L):
                        idx = sidx_v[j, pl.ds(v * L, L)]
                        plsc.addupdate_scatter(cnt, [idx], ones_v)

                @pl.when(i < NBS // NBUF - 1)
                def _():
                    for b in range(NBUF):
                        wait_s(b)
                        start_g(NBUF * (i + 1) + b, b)
                return carry
            lax.fori_loop(0, NBS // NBUF, body, 0)
            for b in range(NBUF):
                wait_s(b)

    def reduce_counts(cnt, rows_per_sub):
        pltpu.sync_copy(cnt, cstage.at[sid, pl.ds(0, cnt.shape[0])])
        plsc.subcore_barrier()
        pltpu.sync_copy(
            cstage.at[pl.ds(0, NS), pl.ds(sid * rows_per_sub, rows_per_sub)],
            redbuf.at[pl.ds(0, NS), pl.ds(0, rows_per_sub)])

        def rsum(v, carry):
            acc16 = redbuf[0, pl.ds(v * L, L)]
            for k in range(1, NS):
                acc16 = acc16 + redbuf[k, pl.ds(v * L, L)]
            cred[pl.ds(v * L, L)] = acc16
            return carry
        lax.fori_loop(0, rows_per_sub // L, rsum, 0)

    run_pass(xp2_hbm, hrg_hbm.at[cid, sid], hc_hbm.at[sid], yacc, cnt_e)
    plsc.subcore_barrier()
    reduce_counts(cnt_e, ER)

    for chunk in range(ER // 40):
        rows = sid * ER + chunk * 40
        pltpu.sync_copy(yacc.at[pl.ds(rows, 40)], sbuf)

        def yrow(r, carry):
            ridx = jnp.broadcast_to(chunk * 40 + r, (L,)).astype(jnp.int32)
            cnt = plsc.load_gather(cred, [ridx])
            pos = cnt > 0.0
            norm = 1.0 / jnp.where(pos, cnt, 1.0)
            for j in range(NV):
                sbuf[r, pl.ds(j * L, L)] = sbuf[r, pl.ds(j * L, L)] * norm
            return carry
        lax.fori_loop(0, 40, yrow, 0)
        pltpu.sync_copy(sbuf, yaug_hbm.at[pl.ds(cid * E_PAD + rows, 40)])
    plsc.subcore_barrier()

    run_pass(yaug_hbm, hcg_hbm.at[cid, sid], hr_hbm.at[sid], xacc, cnt_n)
    plsc.subcore_barrier()
    reduce_counts(cnt_n, NR)

    for chunk in range(NR // 40):
        rows = sid * NR + chunk * 40
        pltpu.sync_copy(xacc.at[pl.ds(rows, 40)], sbuf)

        def xrow(r, carry):
            ridx = jnp.broadcast_to(chunk * 40 + r, (L,)).astype(jnp.int32)
            cnt = plsc.load_gather(cred, [ridx])
            pos = cnt > 0.0
            norm = 1.0 / jnp.where(pos, cnt, 1.0)
            norm = jnp.where(pos, norm, 0.0)
            for j in range(NV):
                pbuf[r, pl.ds(j * L, L)] = (
                    sbuf[r, pl.ds(j * L, L)] * norm + bias_v[pl.ds(j * L, L)])
            return carry
        lax.fori_loop(0, 40, xrow, 0)
        pltpu.sync_copy(
            pbuf, out_hbm.at[pl.ds(rows, 40), pl.ds(cid * DH, DH)])


def kernel(X, X_target, theta, bias, H_row, H_col):
    del X_target
    x_pad = jnp.zeros((N_PAD, D), jnp.float32).at[:N].set(X)
    xp = _matmul(x_pad, theta)
    xp2 = jnp.concatenate([xp[:, :DH], xp[:, DH:]], axis=0)

    pad_n = NNZ_PAD - NNZ
    hr = jnp.concatenate(
        [H_row, jnp.full((pad_n,), N_PAD - 1, jnp.int32)]
    ).reshape(NS, NB, BATCH)
    hc = jnp.concatenate(
        [H_col, jnp.full((pad_n,), E_PAD - 1, jnp.int32)]
    ).reshape(NS, NB, BATCH)
    hr_g = jnp.stack([hr, hr + N_PAD])
    hc_g = jnp.stack([hc, hc + E_PAD])

    out, _ = _sc_fused(xp2, hr_g, hc_g, hr, hc, bias)
    return out[:N]


# speedup vs baseline: 1.5905x; 1.1653x over previous
import functools

import jax
import jax.numpy as jnp
from jax import lax
from jax.experimental import pallas as pl
from jax.experimental.pallas import tpu as pltpu
from jax.experimental.pallas import tpu_sc as plsc

N = 10000
E = 5000
NNZ = 320000
D = 128

NC = 2
NS = 16
L = 16
DH = D // 2
W = DH
NV = W // L

N_PAD = 10240
E_PAD = 5120
BATCH = 128
NB = 160
NBS = 20
NSTG = NB // NBS
NBUF = 2
NNZ_PAD = NS * NB * BATCH
ER = E_PAD // NS
NR = N_PAD // NS

_mesh = plsc.VectorSubcoreMesh(
    core_axis_name="c", subcore_axis_name="s", num_cores=NC, num_subcores=NS
)
_sc_params = pltpu.CompilerParams(use_tc_tiling_on_sc=False,
                                  needs_layout_passes=False)


def _mm_body(x_ref, t_ref, o_ref):
    o_ref[...] = jnp.dot(x_ref[...], t_ref[...],
                         preferred_element_type=jnp.float32)


def _matmul(x_pad, theta):
    bm = 256
    return pl.pallas_call(
        _mm_body,
        grid=(N_PAD // bm,),
        in_specs=[
            pl.BlockSpec((bm, D), lambda i: (i, 0)),
            pl.BlockSpec((D, D), lambda i: (0, 0)),
        ],
        out_specs=pl.BlockSpec((bm, D), lambda i: (i, 0)),
        out_shape=jax.ShapeDtypeStruct((N_PAD, D), jnp.float32),
    )(x_pad, theta)


@functools.partial(
    pl.kernel,
    out_type=jax.ShapeDtypeStruct((N_PAD, D), jnp.float32),
    mesh=_mesh,
    scratch_types=[
        pltpu.VMEM((NBS, BATCH), jnp.int32),
        pltpu.VMEM((NBS, BATCH), jnp.int32),
        pltpu.VMEM((NBUF, BATCH, W), jnp.float32),
        pltpu.VMEM((40, W), jnp.float32),
        pltpu.VMEM((40, DH), jnp.float32),
        pltpu.VMEM((DH,), jnp.float32),
        pltpu.VMEM((E_PAD,), jnp.float32),
        pltpu.VMEM((N_PAD,), jnp.float32),
        pltpu.VMEM((NS, NR), jnp.float32),
        pltpu.VMEM((NR,), jnp.float32),
        pltpu.VMEM_SHARED((E_PAD, W), jnp.float32),
        pltpu.VMEM_SHARED((N_PAD, W), jnp.float32),
        pltpu.VMEM_SHARED((NS, N_PAD), jnp.float32),
        pltpu.SemaphoreType.DMA,
        pltpu.SemaphoreType.DMA,
        pltpu.SemaphoreType.DMA,
        pltpu.SemaphoreType.DMA,
    ],
    compiler_params=_sc_params,
)
def _sc_fused(xp2_hbm, hrg_hbm, hr_hbm, hc_hbm, bias_hbm,
              out_hbm,
              gidx_v, sidx_v, buf, sbuf, pbuf, bias_v, cnt_e, cnt_n,
              redbuf, cred, yacc, xacc, cstage,
              sem0, sem1, sem2, sem3):
    cid = lax.axis_index("c")
    sid = lax.axis_index("s")
    gsem = (sem0, sem1)
    ssem = (sem2, sem3)
    ones_v = jnp.full((L,), 1.0, jnp.float32)
    zeros_v = jnp.zeros((L,), jnp.float32)

    def zrow(r, carry):
        for j in range(NV):
            sbuf[r, pl.ds(j * L, L)] = zeros_v
        return carry
    lax.fori_loop(0, 40, zrow, 0)
    for cpy in range(ER // 40):
        pltpu.sync_copy(sbuf, yacc.at[pl.ds(sid * ER + cpy * 40, 40)])
    for cpy in range(NR // 40):
        pltpu.sync_copy(sbuf, xacc.at[pl.ds(sid * NR + cpy * 40, 40)])

    def zv_e(i, carry):
        cnt_e[pl.ds(i * L, L)] = zeros_v
        return carry
    lax.fori_loop(0, E_PAD // L, zv_e, 0)

    def zv_n(i, carry):
        cnt_n[pl.ds(i * L, L)] = zeros_v
        return carry
    lax.fori_loop(0, N_PAD // L, zv_n, 0)

    pltpu.sync_copy(bias_hbm.at[pl.ds(cid * DH, DH)], bias_v)
    plsc.subcore_barrier()

    def run_pass(tab, g_hbm, s_hbm, acc, cnt):
        def start_g(j, b):
            pltpu.async_copy(tab.at[gidx_v.at[j]], buf.at[b], gsem[b])

        def wait_g(b):
            pltpu.make_async_copy(tab.at[gidx_v.at[0]], buf.at[b],
                                  gsem[b]).wait()

        def start_s(j, b):
            pltpu.async_copy(buf.at[b], acc.at[sidx_v.at[j]], ssem[b],
                             add=True)

        def wait_s(b):
            pltpu.make_async_copy(buf.at[b], acc.at[sidx_v.at[0]],
                                  ssem[b]).wait()

        for st in range(NSTG):
            pltpu.sync_copy(g_hbm.at[pl.ds(st * NBS, NBS)], gidx_v)
            pltpu.sync_copy(s_hbm.at[pl.ds(st * NBS, NBS)], sidx_v)
            for b in range(NBUF):
                start_g(b, b)

            def body(i, carry):
                for b in range(NBUF):
                    j = NBUF * i + b
                    wait_g(b)
                    start_s(j, b)
                    for v in range(BATCH // L):
                        idx = sidx_v[j, pl.ds(v * L, L)]
                        plsc.addupdate_scatter(cnt, [idx], ones_v)

                @pl.when(i < NBS // NBUF - 1)
                def _():
                    for b in range(NBUF):
                        wait_s(b)
                        start_g(NBUF * (i + 1) + b, b)
                return carry
            lax.fori_loop(0, NBS // NBUF, body, 0)
            for b in range(NBUF):
                wait_s(b)

    def reduce_counts(cnt, rows_per_sub):
        pltpu.sync_copy(cnt, cstage.at[sid, pl.ds(0, cnt.shape[0])])
        plsc.subcore_barrier()
        pltpu.sync_copy(
            cstage.at[pl.ds(0, NS), pl.ds(sid * rows_per_sub, rows_per_sub)],
            redbuf.at[pl.ds(0, NS), pl.ds(0, rows_per_sub)])

        def rsum(v, carry):
            acc16 = redbuf[0, pl.ds(v * L, L)]
            for k in range(1, NS):
                acc16 = acc16 + redbuf[k, pl.ds(v * L, L)]
            cred[pl.ds(v * L, L)] = acc16
            return carry
        lax.fori_loop(0, rows_per_sub // L, rsum, 0)

    run_pass(xp2_hbm, hrg_hbm.at[cid, sid], hc_hbm.at[sid], yacc, cnt_e)
    plsc.subcore_barrier()
    reduce_counts(cnt_e, ER)

    for chunk in range(ER // 40):
        rows = sid * ER + chunk * 40
        pltpu.sync_copy(yacc.at[pl.ds(rows, 40)], sbuf)

        def yrow(r, carry):
            ridx = jnp.broadcast_to(chunk * 40 + r, (L,)).astype(jnp.int32)
            cnt = plsc.load_gather(cred, [ridx])
            pos = cnt > 0.0
            norm = 1.0 / jnp.where(pos, cnt, 1.0)
            for j in range(NV):
                sbuf[r, pl.ds(j * L, L)] = sbuf[r, pl.ds(j * L, L)] * norm
            return carry
        lax.fori_loop(0, 40, yrow, 0)
        pltpu.sync_copy(sbuf, yacc.at[pl.ds(rows, 40)])
    plsc.subcore_barrier()

    run_pass(yacc, hc_hbm.at[sid], hr_hbm.at[sid], xacc, cnt_n)
    plsc.subcore_barrier()
    reduce_counts(cnt_n, NR)

    for chunk in range(NR // 40):
        rows = sid * NR + chunk * 40
        pltpu.sync_copy(xacc.at[pl.ds(rows, 40)], sbuf)

        def xrow(r, carry):
            ridx = jnp.broadcast_to(chunk * 40 + r, (L,)).astype(jnp.int32)
            cnt = plsc.load_gather(cred, [ridx])
            pos = cnt > 0.0
            norm = 1.0 / jnp.where(pos, cnt, 1.0)
            norm = jnp.where(pos, norm, 0.0)
            for j in range(NV):
                pbuf[r, pl.ds(j * L, L)] = (
                    sbuf[r, pl.ds(j * L, L)] * norm + bias_v[pl.ds(j * L, L)])
            return carry
        lax.fori_loop(0, 40, xrow, 0)
        pltpu.sync_copy(
            pbuf, out_hbm.at[pl.ds(rows, 40), pl.ds(cid * DH, DH)])


def kernel(X, X_target, theta, bias, H_row, H_col):
    del X_target
    x_pad = jnp.zeros((N_PAD, D), jnp.float32).at[:N].set(X)
    xp = _matmul(x_pad, theta)
    xp2 = jnp.concatenate([xp[:, :DH], xp[:, DH:]], axis=0)

    pad_n = NNZ_PAD - NNZ
    hr = jnp.concatenate(
        [H_row, jnp.full((pad_n,), N_PAD - 1, jnp.int32)]
    ).reshape(NS, NB, BATCH)
    hc = jnp.concatenate(
        [H_col, jnp.full((pad_n,), E_PAD - 1, jnp.int32)]
    ).reshape(NS, NB, BATCH)
    hr_g = jnp.stack([hr, hr + N_PAD])

    out = _sc_fused(xp2, hr_g, hr, hc, bias)
    return out[:N]


# speedup vs baseline: 2.2819x; 1.4347x over previous
import functools

import jax
import jax.numpy as jnp
from jax import lax
from jax.experimental import pallas as pl
from jax.experimental.pallas import tpu as pltpu
from jax.experimental.pallas import tpu_sc as plsc

N = 10000
E = 5000
NNZ = 320000
D = 128

NC = 2
NS = 16
L = 16
DH = D // 2
W = DH
NV = W // L

N_PAD = 10240
E_PAD = 5120
BATCH = 128
NB = 160
NBS = 20
NSTG = NB // NBS
NBUF = 2
NNZ_PAD = NS * NB * BATCH
ER = E_PAD // NS
NR = N_PAD // NS

_mesh = plsc.VectorSubcoreMesh(
    core_axis_name="c", subcore_axis_name="s", num_cores=NC, num_subcores=NS
)
_sc_params = pltpu.CompilerParams(use_tc_tiling_on_sc=False,
                                  needs_layout_passes=False)


def _mm_body(x_ref, t_ref, o_ref):
    o_ref[...] = jnp.dot(x_ref[...], t_ref[...],
                         preferred_element_type=jnp.float32)


def _matmul(x_pad, theta):
    bm = 256
    return pl.pallas_call(
        _mm_body,
        grid=(N_PAD // bm,),
        in_specs=[
            pl.BlockSpec((bm, D), lambda i: (i, 0)),
            pl.BlockSpec((D, D), lambda i: (0, 0)),
        ],
        out_specs=pl.BlockSpec((bm, D), lambda i: (i, 0)),
        out_shape=jax.ShapeDtypeStruct((N_PAD, D), jnp.float32),
    )(x_pad, theta)


@functools.partial(
    pl.kernel,
    out_type=jax.ShapeDtypeStruct((N_PAD, D), jnp.float32),
    mesh=_mesh,
    scratch_types=[
        pltpu.VMEM((NBS, BATCH), jnp.int32),
        pltpu.VMEM((NBS, BATCH), jnp.int32),
        pltpu.VMEM((NBUF, BATCH, W), jnp.float32),
        pltpu.VMEM((40, W), jnp.float32),
        pltpu.VMEM((40, DH), jnp.float32),
        pltpu.VMEM((DH,), jnp.float32),
        pltpu.VMEM((E_PAD,), jnp.float32),
        pltpu.VMEM((N_PAD,), jnp.float32),
        pltpu.VMEM((NS, NR), jnp.float32),
        pltpu.VMEM((NR,), jnp.float32),
        pltpu.VMEM_SHARED((E_PAD, W), jnp.float32),
        pltpu.VMEM_SHARED((N_PAD, W), jnp.float32),
        pltpu.VMEM_SHARED((NS, N_PAD), jnp.float32),
        pltpu.SemaphoreType.DMA,
        pltpu.SemaphoreType.DMA,
        pltpu.SemaphoreType.DMA,
        pltpu.SemaphoreType.DMA,
    ],
    compiler_params=_sc_params,
)
def _sc_fused(xp2_hbm, hr_hbm, hc_hbm, bias_hbm,
              out_hbm,
              gidx_v, sidx_v, buf, sbuf, pbuf, bias_v, cnt_e, cnt_n,
              redbuf, cred, yacc, xacc, cstage,
              sem0, sem1, sem2, sem3):
    cid = lax.axis_index("c")
    sid = lax.axis_index("s")
    gsem = (sem0, sem1)
    ssem = (sem2, sem3)
    ones_v = jnp.full((L,), 1.0, jnp.float32)
    zeros_v = jnp.zeros((L,), jnp.float32)

    def zrow(r, carry):
        for j in range(NV):
            sbuf[r, pl.ds(j * L, L)] = zeros_v
        return carry
    lax.fori_loop(0, 40, zrow, 0)
    for cpy in range(ER // 40):
        pltpu.sync_copy(sbuf, yacc.at[pl.ds(sid * ER + cpy * 40, 40)])
    pltpu.sync_copy(xp2_hbm.at[pl.ds(cid * N_PAD + sid * NR, NR)],
                    xacc.at[pl.ds(sid * NR, NR)])

    def zv_e(i, carry):
        cnt_e[pl.ds(i * L, L)] = zeros_v
        return carry
    lax.fori_loop(0, E_PAD // L, zv_e, 0)

    def zv_n(i, carry):
        cnt_n[pl.ds(i * L, L)] = zeros_v
        return carry
    lax.fori_loop(0, N_PAD // L, zv_n, 0)

    pltpu.sync_copy(bias_hbm.at[pl.ds(cid * DH, DH)], bias_v)
    plsc.subcore_barrier()

    def run_pass(tab, g_hbm, s_hbm, acc, cnt):
        def start_g(j, b):
            pltpu.async_copy(tab.at[gidx_v.at[j]], buf.at[b], gsem[b])

        def wait_g(b):
            pltpu.make_async_copy(tab.at[gidx_v.at[0]], buf.at[b],
                                  gsem[b]).wait()

        def start_s(j, b):
            pltpu.async_copy(buf.at[b], acc.at[sidx_v.at[j]], ssem[b],
                             add=True)

        def wait_s(b):
            pltpu.make_async_copy(buf.at[b], acc.at[sidx_v.at[0]],
                                  ssem[b]).wait()

        for st in range(NSTG):
            pltpu.sync_copy(g_hbm.at[pl.ds(st * NBS, NBS)], gidx_v)
            pltpu.sync_copy(s_hbm.at[pl.ds(st * NBS, NBS)], sidx_v)
            for b in range(NBUF):
                start_g(b, b)

            def body(i, carry):
                for b in range(NBUF):
                    j = NBUF * i + b
                    wait_g(b)
                    start_s(j, b)
                    for v in range(BATCH // L):
                        idx = sidx_v[j, pl.ds(v * L, L)]
                        plsc.addupdate_scatter(cnt, [idx], ones_v)

                @pl.when(i < NBS // NBUF - 1)
                def _():
                    for b in range(NBUF):
                        wait_s(b)
                        start_g(NBUF * (i + 1) + b, b)
                return carry
            lax.fori_loop(0, NBS // NBUF, body, 0)
            for b in range(NBUF):
                wait_s(b)

    def reduce_counts(cnt, rows_per_sub):
        pltpu.sync_copy(cnt, cstage.at[sid, pl.ds(0, cnt.shape[0])])
        plsc.subcore_barrier()
        pltpu.sync_copy(
            cstage.at[pl.ds(0, NS), pl.ds(sid * rows_per_sub, rows_per_sub)],
            redbuf.at[pl.ds(0, NS), pl.ds(0, rows_per_sub)])

        def rsum(v, carry):
            acc16 = redbuf[0, pl.ds(v * L, L)]
            for k in range(1, NS):
                acc16 = acc16 + redbuf[k, pl.ds(v * L, L)]
            cred[pl.ds(v * L, L)] = acc16
            return carry
        lax.fori_loop(0, rows_per_sub // L, rsum, 0)

    run_pass(xacc, hr_hbm.at[sid], hc_hbm.at[sid], yacc, cnt_e)
    plsc.subcore_barrier()
    for cpy in range(NR // 40):
        pltpu.sync_copy(sbuf, xacc.at[pl.ds(sid * NR + cpy * 40, 40)])
    reduce_counts(cnt_e, ER)

    for chunk in range(ER // 40):
        rows = sid * ER + chunk * 40
        pltpu.sync_copy(yacc.at[pl.ds(rows, 40)], sbuf)

        def yrow(r, carry):
            ridx = jnp.broadcast_to(chunk * 40 + r, (L,)).astype(jnp.int32)
            cnt = plsc.load_gather(cred, [ridx])
            pos = cnt > 0.0
            norm = 1.0 / jnp.where(pos, cnt, 1.0)
            for j in range(NV):
                sbuf[r, pl.ds(j * L, L)] = sbuf[r, pl.ds(j * L, L)] * norm
            return carry
        lax.fori_loop(0, 40, yrow, 0)
        pltpu.sync_copy(sbuf, yacc.at[pl.ds(rows, 40)])
    plsc.subcore_barrier()

    run_pass(yacc, hc_hbm.at[sid], hr_hbm.at[sid], xacc, cnt_n)
    plsc.subcore_barrier()
    reduce_counts(cnt_n, NR)

    for chunk in range(NR // 40):
        rows = sid * NR + chunk * 40
        pltpu.sync_copy(xacc.at[pl.ds(rows, 40)], sbuf)

        def xrow(r, carry):
            ridx = jnp.broadcast_to(chunk * 40 + r, (L,)).astype(jnp.int32)
            cnt = plsc.load_gather(cred, [ridx])
            pos = cnt > 0.0
            norm = 1.0 / jnp.where(pos, cnt, 1.0)
            norm = jnp.where(pos, norm, 0.0)
            for j in range(NV):
                pbuf[r, pl.ds(j * L, L)] = (
                    sbuf[r, pl.ds(j * L, L)] * norm + bias_v[pl.ds(j * L, L)])
            return carry
        lax.fori_loop(0, 40, xrow, 0)
        pltpu.sync_copy(
            pbuf, out_hbm.at[pl.ds(rows, 40), pl.ds(cid * DH, DH)])


def kernel(X, X_target, theta, bias, H_row, H_col):
    del X_target
    x_pad = jnp.zeros((N_PAD, D), jnp.float32).at[:N].set(X)
    xp = _matmul(x_pad, theta)
    xp2 = jnp.concatenate([xp[:, :DH], xp[:, DH:]], axis=0)

    pad_n = NNZ_PAD - NNZ
    hr = jnp.concatenate(
        [H_row, jnp.full((pad_n,), N_PAD - 1, jnp.int32)]
    ).reshape(NS, NB, BATCH)
    hc = jnp.concatenate(
        [H_col, jnp.full((pad_n,), E_PAD - 1, jnp.int32)]
    ).reshape(NS, NB, BATCH)
    out = _sc_fused(xp2, hr, hc, bias)
    return out[:N]


# speedup vs baseline: 2.3302x; 1.0212x over previous
import functools

import jax
import jax.numpy as jnp
from jax import lax
from jax.experimental import pallas as pl
from jax.experimental.pallas import tpu as pltpu
from jax.experimental.pallas import tpu_sc as plsc

N = 10000
E = 5000
NNZ = 320000
D = 128

NC = 2
NS = 16
L = 16
DH = D // 2
W = DH
NV = W // L

N_PAD = 10240
E_PAD = 5120
BATCH = 128
NB = 160
NBS = 40
NSTG = NB // NBS
NBUF = 2
NNZ_PAD = NS * NB * BATCH
ER = E_PAD // NS
NR = N_PAD // NS

_mesh = plsc.VectorSubcoreMesh(
    core_axis_name="c", subcore_axis_name="s", num_cores=NC, num_subcores=NS
)
_sc_params = pltpu.CompilerParams(use_tc_tiling_on_sc=False,
                                  needs_layout_passes=False)


def _mm_body(x_ref, t_ref, o_ref):
    o_ref[...] = jnp.dot(x_ref[...], t_ref[...],
                         preferred_element_type=jnp.float32)


def _matmul(x_pad, theta):
    bm = 256
    return pl.pallas_call(
        _mm_body,
        grid=(N_PAD // bm,),
        in_specs=[
            pl.BlockSpec((bm, D), lambda i: (i, 0)),
            pl.BlockSpec((D, D), lambda i: (0, 0)),
        ],
        out_specs=pl.BlockSpec((bm, D), lambda i: (i, 0)),
        out_shape=jax.ShapeDtypeStruct((N_PAD, D), jnp.float32),
    )(x_pad, theta)


@functools.partial(
    pl.kernel,
    out_type=jax.ShapeDtypeStruct((N_PAD, D), jnp.float32),
    mesh=_mesh,
    scratch_types=[
        pltpu.VMEM((NBS, BATCH), jnp.int32),
        pltpu.VMEM((NBS, BATCH), jnp.int32),
        pltpu.VMEM((NBUF, BATCH, W), jnp.float32),
        pltpu.VMEM((40, W), jnp.float32),
        pltpu.VMEM((40, DH), jnp.float32),
        pltpu.VMEM((DH,), jnp.float32),
        pltpu.VMEM((E_PAD,), jnp.float32),
        pltpu.VMEM((N_PAD,), jnp.float32),
        pltpu.VMEM((NS, NR), jnp.float32),
        pltpu.VMEM((NR,), jnp.float32),
        pltpu.VMEM_SHARED((E_PAD, W), jnp.float32),
        pltpu.VMEM_SHARED((N_PAD, W), jnp.float32),
        pltpu.VMEM_SHARED((NS, N_PAD), jnp.float32),
        pltpu.SemaphoreType.DMA,
        pltpu.SemaphoreType.DMA,
        pltpu.SemaphoreType.DMA,
        pltpu.SemaphoreType.DMA,
    ],
    compiler_params=_sc_params,
)
def _sc_fused(xp2_hbm, hr_hbm, hc_hbm, bias_hbm,
              out_hbm,
              gidx_v, sidx_v, buf, sbuf, pbuf, bias_v, cnt_e, cnt_n,
              redbuf, cred, yacc, xacc, cstage,
              sem0, sem1, sem2, sem3):
    cid = lax.axis_index("c")
    sid = lax.axis_index("s")
    gsem = (sem0, sem1)
    ssem = (sem2, sem3)
    ones_v = jnp.full((L,), 1.0, jnp.float32)
    zeros_v = jnp.zeros((L,), jnp.float32)

    def zrow(r, carry):
        for j in range(NV):
            sbuf[r, pl.ds(j * L, L)] = zeros_v
        return carry
    lax.fori_loop(0, 40, zrow, 0)
    for cpy in range(ER // 40):
        pltpu.sync_copy(sbuf, yacc.at[pl.ds(sid * ER + cpy * 40, 40)])
    pltpu.sync_copy(xp2_hbm.at[pl.ds(cid * N_PAD + sid * NR, NR)],
                    xacc.at[pl.ds(sid * NR, NR)])

    def zv_e(i, carry):
        cnt_e[pl.ds(i * L, L)] = zeros_v
        return carry
    lax.fori_loop(0, E_PAD // L, zv_e, 0)

    def zv_n(i, carry):
        cnt_n[pl.ds(i * L, L)] = zeros_v
        return carry
    lax.fori_loop(0, N_PAD // L, zv_n, 0)

    pltpu.sync_copy(bias_hbm.at[pl.ds(cid * DH, DH)], bias_v)
    plsc.subcore_barrier()

    def run_pass(tab, g_hbm, s_hbm, acc, cnt):
        def start_g(j, b):
            pltpu.async_copy(tab.at[gidx_v.at[j]], buf.at[b], gsem[b])

        def wait_g(b):
            pltpu.make_async_copy(tab.at[gidx_v.at[0]], buf.at[b],
                                  gsem[b]).wait()

        def start_s(j, b):
            pltpu.async_copy(buf.at[b], acc.at[sidx_v.at[j]], ssem[b],
                             add=True)

        def wait_s(b):
            pltpu.make_async_copy(buf.at[b], acc.at[sidx_v.at[0]],
                                  ssem[b]).wait()

        for st in range(NSTG):
            pltpu.sync_copy(g_hbm.at[pl.ds(st * NBS, NBS)], gidx_v)
            pltpu.sync_copy(s_hbm.at[pl.ds(st * NBS, NBS)], sidx_v)
            for b in range(NBUF):
                start_g(b, b)

            def body(i, carry):
                for b in range(NBUF):
                    j = NBUF * i + b
                    wait_g(b)
                    start_s(j, b)
                    for v in range(BATCH // L):
                        idx = sidx_v[j, pl.ds(v * L, L)]
                        plsc.addupdate_scatter(cnt, [idx], ones_v)

                @pl.when(i < NBS // NBUF - 1)
                def _():
                    for b in range(NBUF):
                        wait_s(b)
                        start_g(NBUF * (i + 1) + b, b)
                return carry
            lax.fori_loop(0, NBS // NBUF, body, 0)
            for b in range(NBUF):
                wait_s(b)

    def reduce_counts(cnt, rows_per_sub):
        pltpu.sync_copy(cnt, cstage.at[sid, pl.ds(0, cnt.shape[0])])
        plsc.subcore_barrier()
        pltpu.sync_copy(
            cstage.at[pl.ds(0, NS), pl.ds(sid * rows_per_sub, rows_per_sub)],
            redbuf.at[pl.ds(0, NS), pl.ds(0, rows_per_sub)])

        def rsum(v, carry):
            acc16 = redbuf[0, pl.ds(v * L, L)]
            for k in range(1, NS):
                acc16 = acc16 + redbuf[k, pl.ds(v * L, L)]
            cred[pl.ds(v * L, L)] = acc16
            return carry
        lax.fori_loop(0, rows_per_sub // L, rsum, 0)

    run_pass(xacc, hr_hbm.at[sid], hc_hbm.at[sid], yacc, cnt_e)
    plsc.subcore_barrier()
    for cpy in range(NR // 40):
        pltpu.sync_copy(sbuf, xacc.at[pl.ds(sid * NR + cpy * 40, 40)])
    reduce_counts(cnt_e, ER)

    for chunk in range(ER // 40):
        rows = sid * ER + chunk * 40
        pltpu.sync_copy(yacc.at[pl.ds(rows, 40)], sbuf)

        def yrow(r, carry):
            ridx = jnp.broadcast_to(chunk * 40 + r, (L,)).astype(jnp.int32)
            cnt = plsc.load_gather(cred, [ridx])
            pos = cnt > 0.0
            norm = 1.0 / jnp.where(pos, cnt, 1.0)
            for j in range(NV):
                sbuf[r, pl.ds(j * L, L)] = sbuf[r, pl.ds(j * L, L)] * norm
            return carry
        lax.fori_loop(0, 40, yrow, 0)
        pltpu.sync_copy(sbuf, yacc.at[pl.ds(rows, 40)])
    plsc.subcore_barrier()

    run_pass(yacc, hc_hbm.at[sid], hr_hbm.at[sid], xacc, cnt_n)
    plsc.subcore_barrier()
    reduce_counts(cnt_n, NR)

    for chunk in range(NR // 40):
        rows = sid * NR + chunk * 40
        pltpu.sync_copy(xacc.at[pl.ds(rows, 40)], sbuf)

        def xrow(r, carry):
            ridx = jnp.broadcast_to(chunk * 40 + r, (L,)).astype(jnp.int32)
            cnt = plsc.load_gather(cred, [ridx])
            pos = cnt > 0.0
            norm = 1.0 / jnp.where(pos, cnt, 1.0)
            norm = jnp.where(pos, norm, 0.0)
            for j in range(NV):
                pbuf[r, pl.ds(j * L, L)] = (
                    sbuf[r, pl.ds(j * L, L)] * norm + bias_v[pl.ds(j * L, L)])
            return carry
        lax.fori_loop(0, 40, xrow, 0)
        pltpu.sync_copy(
            pbuf, out_hbm.at[pl.ds(rows, 40), pl.ds(cid * DH, DH)])


def kernel(X, X_target, theta, bias, H_row, H_col):
    del X_target
    x_pad = jnp.zeros((N_PAD, D), jnp.float32).at[:N].set(X)
    xp = _matmul(x_pad, theta)
    xp2 = jnp.concatenate([xp[:, :DH], xp[:, DH:]], axis=0)

    pad_n = NNZ_PAD - NNZ
    hr = jnp.concatenate(
        [H_row, jnp.full((pad_n,), N_PAD - 1, jnp.int32)]
    ).reshape(NS, NB, BATCH)
    hc = jnp.concatenate(
        [H_col, jnp.full((pad_n,), E_PAD - 1, jnp.int32)]
    ).reshape(NS, NB, BATCH)
    out = _sc_fused(xp2, hr, hc, bias)
    return out[:N]
